# baseline (speedup 1.0000x reference)
import functools

import jax
import jax.numpy as jnp
from jax import lax
from jax.experimental import pallas as pl
from jax.experimental.pallas import tpu as pltpu
from jax.experimental.pallas import tpu_sc as plsc

N_USERS_K = 2000
N_ITEMS_K = 8000
N_K = N_USERS_K + N_ITEMS_K
D_K = 128
E_K = 320000
NC = 2
NS = 16
NW = NC * NS
LANES = 16

EPW = E_K // NW
CHUNK = 40
NCHUNKS = EPW // CHUNK
NBUF = 8
NIDX = 8
UNROLL = 8
ROWS_PW = 624
ROWS_TAIL = N_K - NS * ROWS_PW

_GDN = lax.GatherDimensionNumbers(
    offset_dims=(), collapsed_slice_dims=(0,), start_index_map=(0,))


def _lane_splat(vec16, lane):
    idx = jnp.full((LANES, 1), lane, dtype=jnp.int32)
    return lax.gather(vec16, idx, _GDN, slice_sizes=(1,),
                      mode=lax.GatherScatterMode.PROMISE_IN_BOUNDS)


def _sc_body(combo_hbm, x_hbm, out_hbm,
             agg_sh, combo_bufs, rows_bufs, isems, gsems, ssems):
    c = lax.axis_index("c")
    s = lax.axis_index("s")
    wid = c * NS + s

    def _idx_start(i, bi):
        pltpu.async_copy(combo_hbm.at[wid * NCHUNKS + i], combo_bufs[bi],
                         isems[bi])

    def _idx_wait(i, bi):
        pltpu.make_async_copy(combo_hbm.at[wid * NCHUNKS + i], combo_bufs[bi],
                              isems[bi]).wait()

    def _gather_start(b, bi):
        pltpu.async_copy(x_hbm.at[combo_bufs[bi].at[1]], rows_bufs[b], gsems[b])

    def _gather_wait(b, bi):
        pltpu.make_async_copy(x_hbm.at[combo_bufs[bi].at[1]], rows_bufs[b],
                              gsems[b]).wait()

    def _scatter_start(b, bi):
        pltpu.async_copy(rows_bufs[b], agg_sh.at[combo_bufs[bi].at[0]],
                         ssems[b], add=True)

    def _scatter_wait(b, bi):
        pltpu.make_async_copy(rows_bufs[b], agg_sh.at[combo_bufs[bi].at[0]],
                              ssems[b]).wait()

    def _scale(b, bi):
        rows = rows_bufs[b]
        cv = combo_bufs[bi]

        def _group(g, _):
            w16 = lax.bitcast_convert_type(cv[2, pl.ds(g * LANES, LANES)],
                                           jnp.float32)
            for e16 in range(LANES):
                splat = _lane_splat(w16, e16)
                e = g * LANES + e16
                for db in range(D_K // LANES):
                    sl = pl.ds(db * LANES, LANES)
                    rows[e, sl] = rows[e, sl] * splat
            return 0
        lax.fori_loop(0, CHUNK // LANES, _group, 0)
        tail = CHUNK % LANES
        if tail:
            w16 = lax.bitcast_convert_type(
                cv[2, pl.ds(CHUNK - LANES, LANES)], jnp.float32)
            for t in range(tail):
                splat = _lane_splat(w16, LANES - tail + t)
                e = CHUNK - tail + t
                for db in range(D_K // LANES):
                    sl = pl.ds(db * LANES, LANES)
                    rows[e, sl] = rows[e, sl] * splat

    for j in range(5):
        _idx_start(j, j)
    for j in range(3):
        _idx_wait(j, j)
        _gather_start(j, j)

    zrows = rows_bufs[NBUF - 1]

    def _zfill(r, _):
        for db in range(D_K // LANES):
            zrows[r, pl.ds(db * LANES, LANES)] = jnp.zeros((LANES,), jnp.float32)
        return 0
    lax.fori_loop(0, CHUNK, _zfill, 0)
    for k in range(ROWS_PW // CHUNK):
        pltpu.sync_copy(zrows, agg_sh.at[pl.ds(s * ROWS_PW + k * CHUNK, CHUNK)])
    rem = ROWS_PW % CHUNK
    pltpu.sync_copy(zrows.at[pl.ds(0, rem)],
                    agg_sh.at[pl.ds(s * ROWS_PW + (ROWS_PW - rem), rem)])

    @pl.when(s == NS - 1)
    def _zero_tail():
        pltpu.sync_copy(zrows.at[pl.ds(0, ROWS_TAIL)],
                        agg_sh.at[pl.ds(NS * ROWS_PW, ROWS_TAIL)])
    plsc.subcore_barrier()

    def _step(i, b, bi):
        @pl.when(i - 3 >= 0)
        def _drain():
            _scatter_wait((b + NBUF - 3) % NBUF, (bi + NIDX - 3) % NIDX)

        @pl.when(i + 5 <= NCHUNKS - 1)
        def _prefetch():
            _idx_start(i + 5, (bi + 5) % NIDX)

        @pl.when(i + 3 <= NCHUNKS - 1)
        def _gather_next():
            _idx_wait(i + 3, (bi + 3) % NIDX)
            _gather_start((b + 3) % NBUF, (bi + 3) % NIDX)

        _gather_wait(b, bi)
        _scale(b, bi)
        _scatter_start(b, bi)

    def _outer(o, _):
        for u in range(UNROLL):
            _step(o * UNROLL + u, u % NBUF, u % NIDX)
        return 0
    nloop = (NCHUNKS // UNROLL) * UNROLL
    lax.fori_loop(0, NCHUNKS // UNROLL, _outer, 0)
    for i in range(nloop, NCHUNKS):
        _step(i, i % NBUF, i % NIDX)

    for i in range(NCHUNKS - 3, NCHUNKS):
        _scatter_wait(i % NBUF, i % NIDX)

    plsc.subcore_barrier()
    pltpu.sync_copy(agg_sh.at[pl.ds(s * ROWS_PW, ROWS_PW)],
                    out_hbm.at[c, pl.ds(s * ROWS_PW, ROWS_PW)])

    @pl.when(s == NS - 1)
    def _copy_tail():
        pltpu.sync_copy(agg_sh.at[pl.ds(NS * ROWS_PW, ROWS_TAIL)],
                        out_hbm.at[c, pl.ds(NS * ROWS_PW, ROWS_TAIL)])


@jax.jit
def _sc_segment_sum(combo, x):
    mesh = plsc.VectorSubcoreMesh(core_axis_name="c", subcore_axis_name="s",
                                  num_cores=NC, num_subcores=NS)
    return pl.kernel(
        _sc_body,
        out_type=jax.ShapeDtypeStruct((NC, N_K, D_K), jnp.float32),
        mesh=mesh,
        scratch_types=[
            pltpu.VMEM_SHARED((N_K, D_K), jnp.float32),
            [pltpu.VMEM((3, CHUNK), jnp.int32) for _ in range(NIDX)],
            [pltpu.VMEM((CHUNK, D_K), jnp.float32) for _ in range(NBUF)],
            [pltpu.SemaphoreType.DMA for _ in range(NIDX)],
            [pltpu.SemaphoreType.DMA for _ in range(NBUF)],
            [pltpu.SemaphoreType.DMA for _ in range(NBUF)],
        ],
    )(combo, x)


def _tc_layer_body(scale, x_ref, p_ref, w1_ref, b1_ref, w2_ref, b2_ref,
                   acc_ref, h_out, acc_out):
    agg = p_ref[0] + p_ref[1]
    xx = x_ref[...]
    h = (jnp.dot(xx + agg, w1_ref[...], preferred_element_type=jnp.float32)
         + jnp.dot(xx * agg, w2_ref[...], preferred_element_type=jnp.float32)
         + b1_ref[...] + b2_ref[...])
    h = jnp.where(h >= 0, h, 0.2 * h)
    nrm = jnp.maximum(jnp.sqrt(jnp.sum(h * h, axis=1, keepdims=True)), 1e-12)
    h = h / nrm
    h_out[...] = h
    acc_out[...] = (acc_ref[...] + h) * scale


@functools.partial(jax.jit, static_argnames=("scale",))
def _tc_layer(x, partials, w1, b1, w2, b2, acc, scale):
    blk = 2000
    grid = (N_K // blk,)
    return pl.pallas_call(
        functools.partial(_tc_layer_body, scale),
        grid=grid,
        in_specs=[
            pl.BlockSpec((blk, D_K), lambda i: (i, 0)),
            pl.BlockSpec((NC, blk, D_K), lambda i: (0, i, 0)),
            pl.BlockSpec((D_K, D_K), lambda i: (0, 0)),
            pl.BlockSpec((1, D_K), lambda i: (0, 0)),
            pl.BlockSpec((D_K, D_K), lambda i: (0, 0)),
            pl.BlockSpec((1, D_K), lambda i: (0, 0)),
            pl.BlockSpec((blk, D_K), lambda i: (i, 0)),
        ],
        out_specs=[
            pl.BlockSpec((blk, D_K), lambda i: (i, 0)),
            pl.BlockSpec((blk, D_K), lambda i: (i, 0)),
        ],
        out_shape=[
            jax.ShapeDtypeStruct((N_K, D_K), jnp.float32),
            jax.ShapeDtypeStruct((N_K, D_K), jnp.float32),
        ],
    )(x, partials, w1, b1.reshape(1, D_K), w2, b2.reshape(1, D_K), acc)


def kernel(edge_index, edge_weight, user_emb, item_emb,
           W1_0, b1_0, W2_0, b2_0,
           W1_1, b1_1, W2_1, b2_1,
           W1_2, b1_2, W2_2, b2_2):
    layers = [(W1_0, b1_0, W2_0, b2_0),
              (W1_1, b1_1, W2_1, b2_1),
              (W1_2, b1_2, W2_2, b2_2)]
    x = jnp.concatenate([user_emb, item_emb], axis=0)
    combo = jnp.stack(
        [edge_index[0].reshape(NW * NCHUNKS, CHUNK),
         edge_index[1].reshape(NW * NCHUNKS, CHUNK),
         lax.bitcast_convert_type(edge_weight, jnp.int32).reshape(
             NW * NCHUNKS, CHUNK)],
        axis=1)
    acc = x
    n_layers = len(layers)
    for l, (w1, b1, w2, b2) in enumerate(layers):
        partials = _sc_segment_sum(combo, x)
        scale = 1.0 / (n_layers + 1) if l == n_layers - 1 else 1.0
        x, acc = _tc_layer(x, partials, w1, b1, w2, b2, acc, scale)
    return acc[:N_USERS_K], acc[N_USERS_K:]


# speedup vs baseline: 1.1133x; 1.1133x over previous
import functools

import jax
import jax.numpy as jnp
from jax import lax
from jax.experimental import pallas as pl
from jax.experimental.pallas import tpu as pltpu
from jax.experimental.pallas import tpu_sc as plsc

N_USERS_K = 2000
N_ITEMS_K = 8000
N_K = N_USERS_K + N_ITEMS_K
D_K = 128
E_K = 320000
NC = 2
NS = 16
NW = NC * NS
LANES = 16

EPW = E_K // NW
CHUNK = 80
NCHUNKS = EPW // CHUNK
NBUF = 4
NIDX = 8
UNROLL = 8
ROWS_PW = 624
ROWS_TAIL = N_K - NS * ROWS_PW

_GDN = lax.GatherDimensionNumbers(
    offset_dims=(), collapsed_slice_dims=(0,), start_index_map=(0,))


def _lane_splat(vec16, lane):
    idx = jnp.full((LANES, 1), lane, dtype=jnp.int32)
    return lax.gather(vec16, idx, _GDN, slice_sizes=(1,),
                      mode=lax.GatherScatterMode.PROMISE_IN_BOUNDS)


def _sc_body(combo_hbm, x_hbm, out_hbm,
             agg_sh, combo_bufs, rows_bufs, isems, gsems, ssems):
    c = lax.axis_index("c")
    s = lax.axis_index("s")
    wid = c * NS + s

    def _idx_start(i, bi):
        pltpu.async_copy(combo_hbm.at[wid * NCHUNKS + i], combo_bufs[bi],
                         isems[bi])

    def _idx_wait(i, bi):
        pltpu.make_async_copy(combo_hbm.at[wid * NCHUNKS + i], combo_bufs[bi],
                              isems[bi]).wait()

    def _gather_start(b, bi):
        pltpu.async_copy(x_hbm.at[combo_bufs[bi].at[1]], rows_bufs[b], gsems[b])

    def _gather_wait(b, bi):
        pltpu.make_async_copy(x_hbm.at[combo_bufs[bi].at[1]], rows_bufs[b],
                              gsems[b]).wait()

    def _scatter_start(b, bi):
        pltpu.async_copy(rows_bufs[b], agg_sh.at[combo_bufs[bi].at[0]],
                         ssems[b], add=True)

    def _scatter_wait(b, bi):
        pltpu.make_async_copy(rows_bufs[b], agg_sh.at[combo_bufs[bi].at[0]],
                              ssems[b]).wait()

    def _scale(b, bi):
        rows = rows_bufs[b]
        cv = combo_bufs[bi]

        def _group(g, _):
            w16 = lax.bitcast_convert_type(cv[2, pl.ds(g * LANES, LANES)],
                                           jnp.float32)
            for e16 in range(LANES):
                splat = _lane_splat(w16, e16)
                e = g * LANES + e16
                for db in range(D_K // LANES):
                    sl = pl.ds(db * LANES, LANES)
                    rows[e, sl] = rows[e, sl] * splat
            return 0
        lax.fori_loop(0, CHUNK // LANES, _group, 0)

    _idx_start(0, 0)
    _idx_start(1, 1)
    _idx_start(2, 2)
    _idx_wait(0, 0)
    _gather_start(0, 0)
    _idx_wait(1, 1)
    _gather_start(1, 1)

    zrows = rows_bufs[3]

    def _zfill(r, _):
        for db in range(D_K // LANES):
            zrows[r, pl.ds(db * LANES, LANES)] = jnp.zeros((LANES,), jnp.float32)
        return 0
    lax.fori_loop(0, CHUNK, _zfill, 0)
    for k in range(ROWS_PW // CHUNK):
        pltpu.sync_copy(zrows, agg_sh.at[pl.ds(s * ROWS_PW + k * CHUNK, CHUNK)])
    rem = ROWS_PW % CHUNK
    pltpu.sync_copy(zrows.at[pl.ds(0, rem)],
                    agg_sh.at[pl.ds(s * ROWS_PW + (ROWS_PW - rem), rem)])

    @pl.when(s == NS - 1)
    def _zero_tail():
        pltpu.sync_copy(zrows.at[pl.ds(0, ROWS_TAIL)],
                        agg_sh.at[pl.ds(NS * ROWS_PW, ROWS_TAIL)])
    plsc.subcore_barrier()

    def _step(i, b, bi):
        @pl.when(i - 2 >= 0)
        def _drain():
            _scatter_wait((b + 2) % NBUF, (bi + NIDX - 2) % NIDX)

        @pl.when(i + 3 <= NCHUNKS - 1)
        def _prefetch():
            _idx_start(i + 3, (bi + 3) % NIDX)

        @pl.when(i + 2 <= NCHUNKS - 1)
        def _gather_next():
            _idx_wait(i + 2, (bi + 2) % NIDX)
            _gather_start((b + 2) % NBUF, (bi + 2) % NIDX)

        _gather_wait(b, bi)
        _scale(b, bi)
        _scatter_start(b, bi)

    def _outer(o, _):
        for u in range(UNROLL):
            _step(o * UNROLL + u, u % NBUF, u % NIDX)
        return 0
    nloop = (NCHUNKS // UNROLL) * UNROLL
    lax.fori_loop(0, NCHUNKS // UNROLL, _outer, 0)
    for i in range(nloop, NCHUNKS):
        _step(i, i % NBUF, i % NIDX)

    for i in range(NCHUNKS - 2, NCHUNKS):
        _scatter_wait(i % NBUF, i % NIDX)

    plsc.subcore_barrier()
    pltpu.sync_copy(agg_sh.at[pl.ds(s * ROWS_PW, ROWS_PW)],
                    out_hbm.at[c, pl.ds(s * ROWS_PW, ROWS_PW)])

    @pl.when(s == NS - 1)
    def _copy_tail():
        pltpu.sync_copy(agg_sh.at[pl.ds(NS * ROWS_PW, ROWS_TAIL)],
                        out_hbm.at[c, pl.ds(NS * ROWS_PW, ROWS_TAIL)])


@jax.jit
def _sc_segment_sum(combo, x):
    mesh = plsc.VectorSubcoreMesh(core_axis_name="c", subcore_axis_name="s",
                                  num_cores=NC, num_subcores=NS)
    return pl.kernel(
        _sc_body,
        out_type=jax.ShapeDtypeStruct((NC, N_K, D_K), jnp.float32),
        mesh=mesh,
        scratch_types=[
            pltpu.VMEM_SHARED((N_K, D_K), jnp.float32),
            [pltpu.VMEM((3, CHUNK), jnp.int32) for _ in range(NIDX)],
            [pltpu.VMEM((CHUNK, D_K), jnp.float32) for _ in range(NBUF)],
            [pltpu.SemaphoreType.DMA for _ in range(NIDX)],
            [pltpu.SemaphoreType.DMA for _ in range(NBUF)],
            [pltpu.SemaphoreType.DMA for _ in range(NBUF)],
        ],
    )(combo, x)


def _tc_layer_body(scale, x_ref, p_ref, w1_ref, b1_ref, w2_ref, b2_ref,
                   acc_ref, h_out, acc_out):
    agg = p_ref[0] + p_ref[1]
    xx = x_ref[...]
    h = (jnp.dot(xx + agg, w1_ref[...], preferred_element_type=jnp.float32)
         + jnp.dot(xx * agg, w2_ref[...], preferred_element_type=jnp.float32)
         + b1_ref[...] + b2_ref[...])
    h = jnp.where(h >= 0, h, 0.2 * h)
    nrm = jnp.maximum(jnp.sqrt(jnp.sum(h * h, axis=1, keepdims=True)), 1e-12)
    h = h / nrm
    h_out[...] = h
    acc_out[...] = (acc_ref[...] + h) * scale


@functools.partial(jax.jit, static_argnames=("scale",))
def _tc_layer(x, partials, w1, b1, w2, b2, acc, scale):
    blk = 2000
    grid = (N_K // blk,)
    return pl.pallas_call(
        functools.partial(_tc_layer_body, scale),
        grid=grid,
        in_specs=[
            pl.BlockSpec((blk, D_K), lambda i: (i, 0)),
            pl.BlockSpec((NC, blk, D_K), lambda i: (0, i, 0)),
            pl.BlockSpec((D_K, D_K), lambda i: (0, 0)),
            pl.BlockSpec((1, D_K), lambda i: (0, 0)),
            pl.BlockSpec((D_K, D_K), lambda i: (0, 0)),
            pl.BlockSpec((1, D_K), lambda i: (0, 0)),
            pl.BlockSpec((blk, D_K), lambda i: (i, 0)),
        ],
        out_specs=[
            pl.BlockSpec((blk, D_K), lambda i: (i, 0)),
            pl.BlockSpec((blk, D_K), lambda i: (i, 0)),
        ],
        out_shape=[
            jax.ShapeDtypeStruct((N_K, D_K), jnp.float32),
            jax.ShapeDtypeStruct((N_K, D_K), jnp.float32),
        ],
    )(x, partials, w1, b1.reshape(1, D_K), w2, b2.reshape(1, D_K), acc)


def kernel(edge_index, edge_weight, user_emb, item_emb,
           W1_0, b1_0, W2_0, b2_0,
           W1_1, b1_1, W2_1, b2_1,
           W1_2, b1_2, W2_2, b2_2):
    layers = [(W1_0, b1_0, W2_0, b2_0),
              (W1_1, b1_1, W2_1, b2_1),
              (W1_2, b1_2, W2_2, b2_2)]
    x = jnp.concatenate([user_emb, item_emb], axis=0)
    combo = jnp.stack(
        [edge_index[0].reshape(NW * NCHUNKS, CHUNK),
         edge_index[1].reshape(NW * NCHUNKS, CHUNK),
         lax.bitcast_convert_type(edge_weight, jnp.int32).reshape(
             NW * NCHUNKS, CHUNK)],
        axis=1)
    acc = x
    n_layers = len(layers)
    for l, (w1, b1, w2, b2) in enumerate(layers):
        partials = _sc_segment_sum(combo, x)
        scale = 1.0 / (n_layers + 1) if l == n_layers - 1 else 1.0
        x, acc = _tc_layer(x, partials, w1, b1, w2, b2, acc, scale)
    return acc[:N_USERS_K], acc[N_USERS_K:]


# speedup vs baseline: 1.1368x; 1.0211x over previous
import functools

import jax
import jax.numpy as jnp
from jax import lax
from jax.experimental import pallas as pl
from jax.experimental.pallas import tpu as pltpu
from jax.experimental.pallas import tpu_sc as plsc

N_USERS_K = 2000
N_ITEMS_K = 8000
N_K = N_USERS_K + N_ITEMS_K
D_K = 128
E_K = 320000
NC = 2
NS = 16
NW = NC * NS
LANES = 16

EPW = E_K // NW
CHUNK = 80
NCHUNKS = EPW // CHUNK
NBUF = 4
NIDX = 8
UNROLL = 8
ROWS_PW = 624
ROWS_TAIL = N_K - NS * ROWS_PW

_GDN = lax.GatherDimensionNumbers(
    offset_dims=(), collapsed_slice_dims=(0,), start_index_map=(0,))


def _lane_splat(vec16, lane):
    idx = jnp.full((LANES, 1), lane, dtype=jnp.int32)
    return lax.gather(vec16, idx, _GDN, slice_sizes=(1,),
                      mode=lax.GatherScatterMode.PROMISE_IN_BOUNDS)


def _sc_body(combo_hbm, x_hbm, out_hbm,
             agg_sh, combo_bufs, rows_bufs, isems, gsems, ssems):
    c = lax.axis_index("c")
    s = lax.axis_index("s")
    wid = c * NS + s

    def _idx_start(i, bi):
        pltpu.async_copy(combo_hbm.at[wid * NCHUNKS + i], combo_bufs[bi],
                         isems[bi])

    def _idx_wait(i, bi):
        pltpu.make_async_copy(combo_hbm.at[wid * NCHUNKS + i], combo_bufs[bi],
                              isems[bi]).wait()

    def _gather_start(b, bi):
        pltpu.async_copy(x_hbm.at[combo_bufs[bi].at[1]], rows_bufs[b], gsems[b])

    def _gather_wait(b, bi):
        pltpu.make_async_copy(x_hbm.at[combo_bufs[bi].at[1]], rows_bufs[b],
                              gsems[b]).wait()

    def _scatter_start(b, bi):
        pltpu.async_copy(rows_bufs[b], agg_sh.at[combo_bufs[bi].at[0]],
                         ssems[b], add=True)

    def _scatter_wait(b, bi):
        pltpu.make_async_copy(rows_bufs[b], agg_sh.at[combo_bufs[bi].at[0]],
                              ssems[b]).wait()

    def _scale(b, bi):
        rows = rows_bufs[b]
        cv = combo_bufs[bi]

        def _group(g, _):
            w16 = lax.bitcast_convert_type(cv[2, pl.ds(g * LANES, LANES)],
                                           jnp.float32)
            for e16 in range(LANES):
                splat = _lane_splat(w16, e16)
                e = g * LANES + e16
                for db in range(D_K // LANES):
                    sl = pl.ds(db * LANES, LANES)
                    rows[e, sl] = rows[e, sl] * splat
            return 0
        lax.fori_loop(0, CHUNK // LANES, _group, 0)

    _idx_start(0, 0)
    _idx_start(1, 1)
    _idx_start(2, 2)
    _idx_wait(0, 0)
    _gather_start(0, 0)
    _idx_wait(1, 1)
    _gather_start(1, 1)

    zrows = rows_bufs[3]

    def _zfill(r, _):
        for db in range(D_K // LANES):
            zrows[r, pl.ds(db * LANES, LANES)] = jnp.zeros((LANES,), jnp.float32)
        return 0
    lax.fori_loop(0, CHUNK, _zfill, 0)
    for k in range(ROWS_PW // CHUNK):
        pltpu.sync_copy(zrows, agg_sh.at[pl.ds(s * ROWS_PW + k * CHUNK, CHUNK)])
    rem = ROWS_PW % CHUNK
    pltpu.sync_copy(zrows.at[pl.ds(0, rem)],
                    agg_sh.at[pl.ds(s * ROWS_PW + (ROWS_PW - rem), rem)])

    @pl.when(s == NS - 1)
    def _zero_tail():
        pltpu.sync_copy(zrows.at[pl.ds(0, ROWS_TAIL)],
                        agg_sh.at[pl.ds(NS * ROWS_PW, ROWS_TAIL)])
    plsc.subcore_barrier()

    def _step(i, b, bi):
        @pl.when(i - 2 >= 0)
        def _drain():
            _scatter_wait((b + 2) % NBUF, (bi + NIDX - 2) % NIDX)

        @pl.when(i + 3 <= NCHUNKS - 1)
        def _prefetch():
            _idx_start(i + 3, (bi + 3) % NIDX)

        @pl.when(i + 2 <= NCHUNKS - 1)
        def _gather_next():
            _idx_wait(i + 2, (bi + 2) % NIDX)
            _gather_start((b + 2) % NBUF, (bi + 2) % NIDX)

        _gather_wait(b, bi)
        _scale(b, bi)
        _scatter_start(b, bi)

    def _outer(o, _):
        for u in range(UNROLL):
            _step(o * UNROLL + u, u % NBUF, u % NIDX)
        return 0
    nloop = (NCHUNKS // UNROLL) * UNROLL
    lax.fori_loop(0, NCHUNKS // UNROLL, _outer, 0)
    for i in range(nloop, NCHUNKS):
        _step(i, i % NBUF, i % NIDX)

    for i in range(NCHUNKS - 2, NCHUNKS):
        _scatter_wait(i % NBUF, i % NIDX)

    plsc.subcore_barrier()
    pltpu.sync_copy(agg_sh.at[pl.ds(s * ROWS_PW, ROWS_PW)],
                    out_hbm.at[c, pl.ds(s * ROWS_PW, ROWS_PW)])

    @pl.when(s == NS - 1)
    def _copy_tail():
        pltpu.sync_copy(agg_sh.at[pl.ds(NS * ROWS_PW, ROWS_TAIL)],
                        out_hbm.at[c, pl.ds(NS * ROWS_PW, ROWS_TAIL)])


@jax.jit
def _sc_segment_sum(combo, x):
    mesh = plsc.VectorSubcoreMesh(core_axis_name="c", subcore_axis_name="s",
                                  num_cores=NC, num_subcores=NS)
    return pl.kernel(
        _sc_body,
        out_type=jax.ShapeDtypeStruct((NC, N_K, D_K), jnp.float32),
        mesh=mesh,
        scratch_types=[
            pltpu.VMEM_SHARED((N_K, D_K), jnp.float32),
            [pltpu.VMEM((3, CHUNK), jnp.int32) for _ in range(NIDX)],
            [pltpu.VMEM((CHUNK, D_K), jnp.float32) for _ in range(NBUF)],
            [pltpu.SemaphoreType.DMA for _ in range(NIDX)],
            [pltpu.SemaphoreType.DMA for _ in range(NBUF)],
            [pltpu.SemaphoreType.DMA for _ in range(NBUF)],
        ],
    )(combo, x)


def _dense_h(x_ref, p_ref, w1_ref, b1_ref, w2_ref, b2_ref):
    agg = p_ref[0] + p_ref[1]
    xx = x_ref[...]
    h = (jnp.dot(xx + agg, w1_ref[...], preferred_element_type=jnp.float32)
         + jnp.dot(xx * agg, w2_ref[...], preferred_element_type=jnp.float32)
         + b1_ref[...] + b2_ref[...])
    h = jnp.where(h >= 0, h, 0.2 * h)
    nrm = jnp.maximum(jnp.sqrt(jnp.sum(h * h, axis=1, keepdims=True)), 1e-12)
    return xx, h / nrm


def _tc_layer_body(first, x_ref, p_ref, w1_ref, b1_ref, w2_ref, b2_ref,
                   *rest):
    if first:
        h_out, acc_out = rest
        xx, h = _dense_h(x_ref, p_ref, w1_ref, b1_ref, w2_ref, b2_ref)
        acc_prev = xx
    else:
        acc_ref, h_out, acc_out = rest
        xx, h = _dense_h(x_ref, p_ref, w1_ref, b1_ref, w2_ref, b2_ref)
        acc_prev = acc_ref[...]
    h_out[...] = h
    acc_out[...] = acc_prev + h


def _tc_last_body(scale, nblk, x_ref, p_ref, w1_ref, b1_ref, w2_ref, b2_ref,
                  acc_ref, user_out, item_out):
    i = pl.program_id(0)
    _, h = _dense_h(x_ref, p_ref, w1_ref, b1_ref, w2_ref, b2_ref)
    a = (acc_ref[...] + h) * scale

    @pl.when(i == 0)
    def _users():
        user_out[...] = a

    @pl.when(i > 0)
    def _items():
        item_out[...] = a


_BLK = 2000


def _common_in_specs():
    return [
        pl.BlockSpec((_BLK, D_K), lambda i: (i, 0)),
        pl.BlockSpec((NC, _BLK, D_K), lambda i: (0, i, 0)),
        pl.BlockSpec((D_K, D_K), lambda i: (0, 0)),
        pl.BlockSpec((1, D_K), lambda i: (0, 0)),
        pl.BlockSpec((D_K, D_K), lambda i: (0, 0)),
        pl.BlockSpec((1, D_K), lambda i: (0, 0)),
    ]


@functools.partial(jax.jit, static_argnames=("first",))
def _tc_layer(x, partials, w1, b1, w2, b2, acc, first=False):
    grid = (N_K // _BLK,)
    in_specs = _common_in_specs()
    args = [x, partials, w1, b1.reshape(1, D_K), w2, b2.reshape(1, D_K)]
    if not first:
        in_specs.append(pl.BlockSpec((_BLK, D_K), lambda i: (i, 0)))
        args.append(acc)
    return pl.pallas_call(
        functools.partial(_tc_layer_body, first),
        grid=grid,
        in_specs=in_specs,
        out_specs=[
            pl.BlockSpec((_BLK, D_K), lambda i: (i, 0)),
            pl.BlockSpec((_BLK, D_K), lambda i: (i, 0)),
        ],
        out_shape=[
            jax.ShapeDtypeStruct((N_K, D_K), jnp.float32),
            jax.ShapeDtypeStruct((N_K, D_K), jnp.float32),
        ],
    )(*args)


@functools.partial(jax.jit, static_argnames=("scale",))
def _tc_layer_last(x, partials, w1, b1, w2, b2, acc, scale):
    grid = (N_K // _BLK,)
    nblk = N_K // _BLK
    in_specs = _common_in_specs()
    in_specs.append(pl.BlockSpec((_BLK, D_K), lambda i: (i, 0)))
    return pl.pallas_call(
        functools.partial(_tc_last_body, scale, nblk),
        grid=grid,
        in_specs=in_specs,
        out_specs=[
            pl.BlockSpec((N_USERS_K, D_K), lambda i: (0, 0)),
            pl.BlockSpec((_BLK, D_K),
                         lambda i: (jnp.maximum(i - 1, 0), 0)),
        ],
        out_shape=[
            jax.ShapeDtypeStruct((N_USERS_K, D_K), jnp.float32),
            jax.ShapeDtypeStruct((N_ITEMS_K, D_K), jnp.float32),
        ],
    )(x, partials, w1, b1.reshape(1, D_K), w2, b2.reshape(1, D_K), acc)


def kernel(edge_index, edge_weight, user_emb, item_emb,
           W1_0, b1_0, W2_0, b2_0,
           W1_1, b1_1, W2_1, b2_1,
           W1_2, b1_2, W2_2, b2_2):
    layers = [(W1_0, b1_0, W2_0, b2_0),
              (W1_1, b1_1, W2_1, b2_1),
              (W1_2, b1_2, W2_2, b2_2)]
    x = jnp.concatenate([user_emb, item_emb], axis=0)
    combo = jnp.stack(
        [edge_index[0].reshape(NW * NCHUNKS, CHUNK),
         edge_index[1].reshape(NW * NCHUNKS, CHUNK),
         lax.bitcast_convert_type(edge_weight, jnp.int32).reshape(
             NW * NCHUNKS, CHUNK)],
        axis=1)
    acc = x
    n_layers = len(layers)
    for l, (w1, b1, w2, b2) in enumerate(layers):
        partials = _sc_segment_sum(combo, x)
        if l == n_layers - 1:
            return _tc_layer_last(x, partials, w1, b1, w2, b2, acc,
                                  1.0 / (n_layers + 1))
        x, acc = _tc_layer(x, partials, w1, b1, w2, b2, acc, first=(l == 0))
